# baseline (speedup 1.0000x reference)
import functools
import math

import jax
import jax.numpy as jnp
from jax import lax
from jax.experimental import pallas as pl
from jax.experimental.pallas import tpu as pltpu
from jax.experimental.pallas import tpu_sc as plsc

NUM_CORES = 2
NUM_SUBCORES = 16
NUM_WORKERS = NUM_CORES * NUM_SUBCORES
LANES = 16
TILE = 128
TPB = 10
BLOCK = TILE * TPB

_C0 = math.sqrt(1.0 / (4.0 * math.pi))
_C1 = math.sqrt(3.0 / (4.0 * math.pi))
_C2M2 = math.sqrt(15.0 / (4.0 * math.pi))
_C20 = 0.25 * math.sqrt(5.0 / math.pi)
_C22 = 0.25 * math.sqrt(15.0 / math.pi)
_C3M3 = math.sqrt(35.0 / (32.0 * math.pi))
_C3M2 = 0.5 * math.sqrt(105.0 / math.pi)
_C3M1 = math.sqrt(21.0 / (32.0 * math.pi))
_C30 = 0.25 * math.sqrt(7.0 / math.pi)
_C32 = 0.25 * math.sqrt(105.0 / math.pi)


def _splat_f(v):
    return jnp.full((LANES,), v, jnp.float32)


def _splat_i(v):
    return jnp.full((LANES,), v, jnp.int32)


def _rsqrt_newton(s2):
    i = lax.bitcast_convert_type(s2, jnp.int32)
    seed = _splat_i(0x5F3759DF) - lax.shift_right_arithmetic(i, _splat_i(1))
    y = lax.bitcast_convert_type(seed, jnp.float32)
    half = _splat_f(0.5) * s2
    three_half = _splat_f(1.5)
    for _ in range(2):
        y = y * (three_half - half * y * y)
    return y


def _sh_coeffs(X, Y, Z):
    X2 = X * X
    Y2 = Y * Y
    Z2 = Z * Z
    XY = X * Y
    one = _splat_f(1.0)
    return [
        _splat_f(_C0),
        _splat_f(-_C1) * Y,
        _splat_f(_C1) * Z,
        _splat_f(-_C1) * X,
        _splat_f(_C2M2) * XY,
        _splat_f(-_C2M2) * (Y * Z),
        _splat_f(_C20) * (_splat_f(3.0) * Z2 - one),
        _splat_f(-_C2M2) * (X * Z),
        _splat_f(_C22) * (X2 - Y2),
        _splat_f(-_C3M3) * Y * (_splat_f(3.0) * X2 - Y2),
        _splat_f(_C3M2) * XY * Z,
        _splat_f(-_C3M1) * Y * (_splat_f(5.0) * Z2 - one),
        _splat_f(_C30) * Z * (_splat_f(5.0) * Z2 - _splat_f(3.0)),
        _splat_f(-_C3M1) * X * (_splat_f(5.0) * Z2 - one),
        _splat_f(_C32) * Z * (X2 - Y2),
        _splat_f(-_C3M3) * X * (X2 - _splat_f(3.0) * Y2),
    ]


def kernel(xyz_data, xyz_query, nn_idx):
    num_edges = nn_idx.shape[0]
    assert num_edges % (TILE * TPB) == 0
    nblocks_total = num_edges // BLOCK
    nfull = nblocks_total // NUM_WORKERS
    nrem = nblocks_total - nfull * NUM_WORKERS
    assert nfull >= 3 and nfull % 2 == 1
    half_words = num_edges * 8

    data_h = xyz_data * jnp.float32(0.5)
    query_h = xyz_query * jnp.float32(0.5)
    xd, yd, zd = (data_h[:, c] for c in range(3))
    xq, yq, zq = (query_h[:, c] for c in range(3))
    idx_q = nn_idx[:, 0]
    idx_d = nn_idx[:, 1]

    mesh = plsc.VectorSubcoreMesh(core_axis_name="c", subcore_axis_name="s")

    scratch = (
        [pltpu.VMEM((BLOCK,), jnp.int32)] * 4
        + [pltpu.VMEM((BLOCK,), jnp.float32)] * 12
        + [pltpu.VMEM((BLOCK * 16,), jnp.float32)] * 2
        + [pltpu.VMEM_SHARED((50000,), jnp.float32)] * 6
        + [pltpu.SemaphoreType.DMA] * 4
    )

    @functools.partial(
        pl.kernel,
        out_type=jax.ShapeDtypeStruct((num_edges * 16,), jnp.float32),
        mesh=mesh,
        scratch_types=scratch,
        compiler_params=pltpu.CompilerParams(
            needs_layout_passes=False, use_tc_tiling_on_sc=False
        ),
    )
    def sc_kernel(
        xd_hbm, yd_hbm, zd_hbm, xq_hbm, yq_hbm, zq_hbm, iq_hbm, id_hbm, out_hbm,
        iq0, iq1, id0, id1,
        xd0, xd1, yd0, yd1, zd0, zd1, xq0, xq1, yq0, yq1, zq0, zq1,
        ov0, ov1,
        sp0, sp1, sp2, sp3, sp4, sp5,
        sg0, sg1, so0, so1,
    ):
        wid = lax.axis_index("s") * NUM_CORES + lax.axis_index("c")
        iq_v = (iq0, iq1)
        id_v = (id0, id1)
        planes = ((xd0, xd1), (yd0, yd1), (zd0, zd1),
                  (xq0, xq1), (yq0, yq1), (zq0, zq1))
        out_v = (ov0, ov1)
        sem_g = (sg0, sg1)
        sem_o = (so0, so1)
        hbm_tables = (xd_hbm, yd_hbm, zd_hbm, xq_hbm, yq_hbm, zq_hbm)
        tables = (sp0, sp1, sp2, sp3, sp4, sp5)
        n_rows = hbm_tables[0].shape[0]
        sid = lax.axis_index("s")
        chunk = 2000
        nchunk_mine = (n_rows // chunk - sid + NUM_SUBCORES - 1) // NUM_SUBCORES

        def stage_body(k, _):
            c0 = (sid + k * NUM_SUBCORES) * chunk
            for hbm_t, sp_t in zip(hbm_tables, tables):
                pltpu.sync_copy(
                    hbm_t.at[pl.ds(c0, chunk)], sp_t.at[pl.ds(c0, chunk)]
                )
            return 0

        lax.fori_loop(0, nchunk_mine, stage_body, 0)
        plsc.subcore_barrier()

        def gather_args(s):
            for t, tab in enumerate(tables):
                idx = id_v[s] if t < 3 else iq_v[s]
                yield tab.at[idx], planes[t][s], sem_g[s]

        def fetch(j, s):
            base = (wid + NUM_WORKERS * j) * BLOCK
            pltpu.sync_copy(iq_hbm.at[pl.ds(base, BLOCK)], iq_v[s])
            pltpu.sync_copy(id_hbm.at[pl.ds(base, BLOCK)], id_v[s])
            for src, dst, sem in gather_args(s):
                pltpu.async_copy(src, dst, sem)

        def drain_gathers(s):
            for src, dst, sem in gather_args(s):
                pltpu.make_async_copy(src, dst, sem).wait()

        def out_chunks(j, s):
            tile0 = (wid + NUM_WORKERS * j) * TPB
            for gg in range(2):
                src = out_v[s].at[pl.ds(gg * (TPB * 1024), TPB * 1024)]
                dst = out_hbm.at[
                    pl.ds(gg * half_words + tile0 * 1024, TPB * 1024)
                ]
                yield src, dst, sem_o[s]

        def put_out(j, s):
            for src, dst, sem in out_chunks(j, s):
                pltpu.async_copy(src, dst, sem)

        def wait_out(j, s):
            for src, dst, sem in out_chunks(j, s):
                pltpu.make_async_copy(src, dst, sem).wait()

        UNROLL = 4

        def compute(s):
            xdv, ydv, zdv = planes[0][s], planes[1][s], planes[2][s]
            xqv, yqv, zqv = planes[3][s], planes[4][s], planes[5][s]
            ov = out_v[s]

            def group(jj):
                sl = pl.ds(jj * LANES, LANES)
                dx = xdv[sl] - xqv[sl]
                dy = ydv[sl] - yqv[sl]
                dz = zdv[sl] - zqv[sl]
                s2 = dx * dx + dy * dy + dz * dz
                rinv = _rsqrt_newton(s2)
                coeffs = _sh_coeffs(dx * rinv, dy * rinv, dz * rinv)
                obase = (jj // 8) * 1024 + (jj % 8) * LANES
                for c in range(16):
                    off = (c // 8) * (TPB * 1024) + (c % 8) * TILE
                    ov[pl.ds(obase + off, LANES)] = coeffs[c]

            def vec_body(j, _):
                for u in range(UNROLL):
                    group(j * UNROLL + u)
                return 0

            lax.fori_loop(0, (BLOCK // LANES) // UNROLL, vec_body, 0)

        def block_step(j, s):
            fetch(j + 1, 1 - s)
            drain_gathers(s)

            @pl.when(j >= 2)
            def _():
                wait_out(j - 2, s)

            compute(s)
            put_out(j, s)

        fetch(0, 0)

        def pair_body(i, _):
            block_step(2 * i, 0)
            block_step(2 * i + 1, 1)
            return 0

        lax.fori_loop(0, (nfull - 1) // 2, pair_body, 0)

        j_tail = nfull - 1
        has_extra = wid < nrem

        @pl.when(has_extra)
        def _():
            fetch(nfull, 1)

        drain_gathers(0)
        wait_out(j_tail - 2, 0)
        compute(0)
        put_out(j_tail, 0)
        wait_out(j_tail - 1, 1)

        @pl.when(has_extra)
        def _():
            drain_gathers(1)
            compute(1)
            put_out(nfull, 1)
            wait_out(nfull, 1)

        wait_out(j_tail, 0)

    out = sc_kernel(xd, yd, zd, xq, yq, zq, idx_q, idx_d)
    out = out.reshape(2, num_edges // TILE, 8, TILE)
    return out.transpose(1, 3, 0, 2).reshape(num_edges, 16)


# speedup vs baseline: 1.0046x; 1.0046x over previous
import functools
import math

import jax
import jax.numpy as jnp
from jax import lax
from jax.experimental import pallas as pl
from jax.experimental.pallas import tpu as pltpu
from jax.experimental.pallas import tpu_sc as plsc

NUM_CORES = 2
NUM_SUBCORES = 16
NUM_WORKERS = NUM_CORES * NUM_SUBCORES
LANES = 16
TILE = 128
TPB = 10
BLOCK = TILE * TPB

_C0 = math.sqrt(1.0 / (4.0 * math.pi))
_C1 = math.sqrt(3.0 / (4.0 * math.pi))
_C2M2 = math.sqrt(15.0 / (4.0 * math.pi))
_C20 = 0.25 * math.sqrt(5.0 / math.pi)
_C22 = 0.25 * math.sqrt(15.0 / math.pi)
_C3M3 = math.sqrt(35.0 / (32.0 * math.pi))
_C3M2 = 0.5 * math.sqrt(105.0 / math.pi)
_C3M1 = math.sqrt(21.0 / (32.0 * math.pi))
_C30 = 0.25 * math.sqrt(7.0 / math.pi)
_C32 = 0.25 * math.sqrt(105.0 / math.pi)


def _splat_f(v):
    return jnp.full((LANES,), v, jnp.float32)


def _splat_i(v):
    return jnp.full((LANES,), v, jnp.int32)


def _rsqrt_newton(s2):
    i = lax.bitcast_convert_type(s2, jnp.int32)
    seed = _splat_i(0x5F3759DF) - lax.shift_right_arithmetic(i, _splat_i(1))
    y = lax.bitcast_convert_type(seed, jnp.float32)
    half = _splat_f(0.5) * s2
    three_half = _splat_f(1.5)
    for _ in range(2):
        y = y * (three_half - half * y * y)
    return y


def _sh_coeffs(X, Y, Z):
    X2 = X * X
    Y2 = Y * Y
    Z2 = Z * Z
    XY = X * Y
    one = _splat_f(1.0)
    return [
        _splat_f(_C0),
        _splat_f(-_C1) * Y,
        _splat_f(_C1) * Z,
        _splat_f(-_C1) * X,
        _splat_f(_C2M2) * XY,
        _splat_f(-_C2M2) * (Y * Z),
        _splat_f(_C20) * (_splat_f(3.0) * Z2 - one),
        _splat_f(-_C2M2) * (X * Z),
        _splat_f(_C22) * (X2 - Y2),
        _splat_f(-_C3M3) * Y * (_splat_f(3.0) * X2 - Y2),
        _splat_f(_C3M2) * XY * Z,
        _splat_f(-_C3M1) * Y * (_splat_f(5.0) * Z2 - one),
        _splat_f(_C30) * Z * (_splat_f(5.0) * Z2 - _splat_f(3.0)),
        _splat_f(-_C3M1) * X * (_splat_f(5.0) * Z2 - one),
        _splat_f(_C32) * Z * (X2 - Y2),
        _splat_f(-_C3M3) * X * (X2 - _splat_f(3.0) * Y2),
    ]


def kernel(xyz_data, xyz_query, nn_idx):
    num_edges = nn_idx.shape[0]
    assert num_edges % (TILE * TPB) == 0
    nblocks_total = num_edges // BLOCK
    nfull = nblocks_total // NUM_WORKERS
    nrem = nblocks_total - nfull * NUM_WORKERS
    assert nfull >= 3 and nfull % 2 == 1
    half_words = num_edges * 8

    data_h = xyz_data * jnp.float32(0.5)
    query_h = xyz_query * jnp.float32(0.5)
    xd, yd, zd = (data_h[:, c] for c in range(3))
    xq, yq, zq = (query_h[:, c] for c in range(3))
    idx_q = nn_idx[:, 0]
    idx_d = nn_idx[:, 1]

    mesh = plsc.VectorSubcoreMesh(core_axis_name="c", subcore_axis_name="s")

    scratch = (
        [pltpu.VMEM((BLOCK,), jnp.int32)] * 4
        + [pltpu.VMEM((BLOCK,), jnp.float32)] * 12
        + [pltpu.VMEM((BLOCK * 16,), jnp.float32)] * 2
        + [pltpu.VMEM_SHARED((50000,), jnp.float32)] * 6
        + [pltpu.SemaphoreType.DMA] * 4
    )

    @functools.partial(
        pl.kernel,
        out_type=jax.ShapeDtypeStruct((num_edges * 16,), jnp.float32),
        mesh=mesh,
        scratch_types=scratch,
        compiler_params=pltpu.CompilerParams(
            needs_layout_passes=False, use_tc_tiling_on_sc=False
        ),
    )
    def sc_kernel(
        xd_hbm, yd_hbm, zd_hbm, xq_hbm, yq_hbm, zq_hbm, iq_hbm, id_hbm, out_hbm,
        iq0, iq1, id0, id1,
        xd0, xd1, yd0, yd1, zd0, zd1, xq0, xq1, yq0, yq1, zq0, zq1,
        ov0, ov1,
        sp0, sp1, sp2, sp3, sp4, sp5,
        sg0, sg1, so0, so1,
    ):
        wid = lax.axis_index("s") * NUM_CORES + lax.axis_index("c")
        iq_v = (iq0, iq1)
        id_v = (id0, id1)
        planes = ((xd0, xd1), (yd0, yd1), (zd0, zd1),
                  (xq0, xq1), (yq0, yq1), (zq0, zq1))
        out_v = (ov0, ov1)
        sem_g = (sg0, sg1)
        sem_o = (so0, so1)
        hbm_tables = (xd_hbm, yd_hbm, zd_hbm, xq_hbm, yq_hbm, zq_hbm)
        tables = (sp0, sp1, sp2, sp3, sp4, sp5)
        n_rows = hbm_tables[0].shape[0]
        sid = lax.axis_index("s")
        chunk = 2000
        nchunk_mine = (n_rows // chunk - sid + NUM_SUBCORES - 1) // NUM_SUBCORES

        def stage_body(k, _):
            c0 = (sid + k * NUM_SUBCORES) * chunk
            for hbm_t, sp_t in zip(hbm_tables, tables):
                pltpu.sync_copy(
                    hbm_t.at[pl.ds(c0, chunk)], sp_t.at[pl.ds(c0, chunk)]
                )
            return 0

        lax.fori_loop(0, nchunk_mine, stage_body, 0)
        plsc.subcore_barrier()

        def gather_args(s):
            for t, tab in enumerate(tables):
                idx = id_v[s] if t < 3 else iq_v[s]
                yield tab.at[idx], planes[t][s], sem_g[s]

        def fetch(j, s):
            base = (wid + NUM_WORKERS * j) * BLOCK
            pltpu.sync_copy(iq_hbm.at[pl.ds(base, BLOCK)], iq_v[s])
            pltpu.sync_copy(id_hbm.at[pl.ds(base, BLOCK)], id_v[s])
            for src, dst, sem in gather_args(s):
                pltpu.async_copy(src, dst, sem)

        def drain_gathers(s):
            for src, dst, sem in gather_args(s):
                pltpu.make_async_copy(src, dst, sem).wait()

        def out_chunks(j, s):
            tile0 = (wid + NUM_WORKERS * j) * TPB
            for gg in range(2):
                src = out_v[s].at[pl.ds(gg * (TPB * 1024), TPB * 1024)]
                dst = out_hbm.at[
                    pl.ds(gg * half_words + tile0 * 1024, TPB * 1024)
                ]
                yield src, dst, sem_o[s]

        def put_out(j, s):
            for src, dst, sem in out_chunks(j, s):
                pltpu.async_copy(src, dst, sem)

        def wait_out(j, s):
            for src, dst, sem in out_chunks(j, s):
                pltpu.make_async_copy(src, dst, sem).wait()

        UNROLL = 2

        def compute(s):
            xdv, ydv, zdv = planes[0][s], planes[1][s], planes[2][s]
            xqv, yqv, zqv = planes[3][s], planes[4][s], planes[5][s]
            ov = out_v[s]

            def group(jj):
                sl = pl.ds(jj * LANES, LANES)
                dx = xdv[sl] - xqv[sl]
                dy = ydv[sl] - yqv[sl]
                dz = zdv[sl] - zqv[sl]
                s2 = dx * dx + dy * dy + dz * dz
                rinv = _rsqrt_newton(s2)
                coeffs = _sh_coeffs(dx * rinv, dy * rinv, dz * rinv)
                obase = (jj // 8) * 1024 + (jj % 8) * LANES
                for c in range(16):
                    off = (c // 8) * (TPB * 1024) + (c % 8) * TILE
                    ov[pl.ds(obase + off, LANES)] = coeffs[c]

            def vec_body(j, _):
                for u in range(UNROLL):
                    group(j * UNROLL + u)
                return 0

            lax.fori_loop(0, (BLOCK // LANES) // UNROLL, vec_body, 0)

        def block_step(j, s):
            fetch(j + 1, 1 - s)
            drain_gathers(s)

            @pl.when(j >= 2)
            def _():
                wait_out(j - 2, s)

            compute(s)
            put_out(j, s)

        fetch(0, 0)

        def pair_body(i, _):
            block_step(2 * i, 0)
            block_step(2 * i + 1, 1)
            return 0

        lax.fori_loop(0, (nfull - 1) // 2, pair_body, 0)

        j_tail = nfull - 1
        has_extra = wid < nrem

        @pl.when(has_extra)
        def _():
            fetch(nfull, 1)

        drain_gathers(0)
        wait_out(j_tail - 2, 0)
        compute(0)
        put_out(j_tail, 0)
        wait_out(j_tail - 1, 1)

        @pl.when(has_extra)
        def _():
            drain_gathers(1)
            compute(1)
            put_out(nfull, 1)
            wait_out(nfull, 1)

        wait_out(j_tail, 0)

    out = sc_kernel(xd, yd, zd, xq, yq, zq, idx_q, idx_d)
    out = out.reshape(2, num_edges // TILE, 8, TILE)
    return out.transpose(1, 3, 0, 2).reshape(num_edges, 16)
